# Initial kernel scaffold; baseline (speedup 1.0000x reference)
import jax
import jax.numpy as jnp
from jax.experimental import pallas as pl


def kernel(input_ids, labels, emb_table, W, b):
    raise NotImplementedError("write your pallas kernel here")


# speedup vs baseline: 11.2590x; 11.2590x over previous
import functools

import jax
import jax.numpy as jnp
from jax import lax
from jax.experimental import pallas as pl
from jax.experimental.pallas import tpu as pltpu
from jax.experimental.pallas import tpu_sc as plsc

_LANES = 128
_NEG = -1e30


def _make_sc_gather(vocab, dim, batch):
    info = plsc.get_sparse_core_info()
    nc, ns = info.num_cores, info.num_subcores
    nw = nc * ns
    assert batch % (8 * nw) == 0
    b_per_w = batch // nw
    mesh = plsc.VectorSubcoreMesh(core_axis_name="c", subcore_axis_name="s")

    @functools.partial(
        pl.kernel,
        mesh=mesh,
        out_type=jax.ShapeDtypeStruct((batch, dim), jnp.float32),
        scratch_types=[
            pltpu.VMEM((b_per_w,), jnp.int32),
            pltpu.VMEM((b_per_w, dim), jnp.float32),
            pltpu.SemaphoreType.DMA,
        ],
        compiler_params=pltpu.CompilerParams(use_tc_tiling_on_sc=False),
    )
    def gather_rows(idx_hbm, table_hbm, out_hbm, idx_v, rows_v, sem):
        wid = lax.axis_index("s") * nc + lax.axis_index("c")
        base = wid * b_per_w
        pltpu.sync_copy(idx_hbm.at[pl.ds(base, b_per_w)], idx_v)
        pltpu.async_copy(table_hbm.at[idx_v], rows_v, sem).wait()
        pltpu.sync_copy(rows_v, out_hbm.at[pl.ds(base, b_per_w)])

    return gather_rows


def _cls_body(rows_ref, wt_ref, b_ref, labels_ref, logits_ref, loss_ref):
    rows = rows_ref[...]
    wt = wt_ref[...]
    logits = (
        jnp.dot(rows, wt, preferred_element_type=jnp.float32) + b_ref[...]
    )
    logits_ref[...] = logits
    batch = rows.shape[0]
    m = jnp.max(logits, axis=1, keepdims=True)
    lse = m[:, 0] + jnp.log(jnp.sum(jnp.exp(logits - m), axis=1))
    lane = lax.broadcasted_iota(jnp.int32, (batch, _LANES), 1)
    picked = jnp.sum(jnp.where(lane == labels_ref[...], logits, 0.0), axis=1)
    loss_ref[0, 0] = jnp.mean(lse - picked)


def kernel(input_ids, labels, emb_table, W, b):
    batch = input_ids.shape[0]
    vocab, dim = emb_table.shape
    num_labels = W.shape[0]

    idx = input_ids[:, 0]
    rows = _make_sc_gather(vocab, dim, batch)(idx, emb_table)

    wt = jnp.zeros((dim, _LANES), jnp.float32).at[:, :num_labels].set(W.T)
    b_pad = jnp.full((1, _LANES), _NEG, jnp.float32).at[0, :num_labels].set(b)

    logits_pad, loss = pl.pallas_call(
        _cls_body,
        out_shape=(
            jax.ShapeDtypeStruct((batch, _LANES), jnp.float32),
            jax.ShapeDtypeStruct((1, 1), jnp.float32),
        ),
        in_specs=[
            pl.BlockSpec(memory_space=pltpu.VMEM),
            pl.BlockSpec(memory_space=pltpu.VMEM),
            pl.BlockSpec(memory_space=pltpu.VMEM),
            pl.BlockSpec(memory_space=pltpu.VMEM),
        ],
        out_specs=(
            pl.BlockSpec(memory_space=pltpu.VMEM),
            pl.BlockSpec(memory_space=pltpu.SMEM),
        ),
    )(rows, wt, b_pad, labels[:, None].astype(jnp.int32))

    return loss[0, 0], logits_pad[:, :num_labels]


# speedup vs baseline: 11.4266x; 1.0149x over previous
import functools

import jax
import jax.numpy as jnp
from jax import lax
from jax.experimental import pallas as pl
from jax.experimental.pallas import tpu as pltpu
from jax.experimental.pallas import tpu_sc as plsc

_LANES = 128
_NEG = -1e30


def _make_sc_gather(vocab2, batch):
    info = plsc.get_sparse_core_info()
    nc, ns = info.num_cores, info.num_subcores
    nw = nc * ns
    assert batch % (8 * nw) == 0
    b_per_w = batch // nw
    mesh = plsc.VectorSubcoreMesh(core_axis_name="c", subcore_axis_name="s")

    @functools.partial(
        pl.kernel,
        mesh=mesh,
        out_type=jax.ShapeDtypeStruct((batch, _LANES), jnp.float32),
        scratch_types=[
            pltpu.VMEM((b_per_w,), jnp.int32),
            pltpu.VMEM((b_per_w, _LANES), jnp.float32),
            pltpu.SemaphoreType.DMA,
        ],
    )
    def gather_rows(idx_hbm, table_hbm, out_hbm, idx_v, rows_v, sem):
        wid = lax.axis_index("s") * nc + lax.axis_index("c")
        base = wid * b_per_w
        pltpu.sync_copy(idx_hbm.at[pl.ds(base, b_per_w)], idx_v)
        pltpu.async_copy(table_hbm.at[idx_v], rows_v, sem).wait()
        pltpu.sync_copy(rows_v, out_hbm.at[pl.ds(base, b_per_w)])

    return gather_rows


def _cls_body(pairs_ref, par_ref, wt_ref, b_ref, labels_ref, logits_ref,
              loss_ref):
    pairs = pairs_ref[...]
    dim = pairs.shape[1] // 2
    rows = jnp.where(par_ref[...] == 0, pairs[:, :dim], pairs[:, dim:])
    wt = wt_ref[...]
    logits = (
        jnp.dot(rows, wt, preferred_element_type=jnp.float32) + b_ref[...]
    )
    logits_ref[...] = logits
    batch = pairs.shape[0]
    m = jnp.max(logits, axis=1, keepdims=True)
    lse = m[:, 0] + jnp.log(jnp.sum(jnp.exp(logits - m), axis=1))
    lane = lax.broadcasted_iota(jnp.int32, (batch, _LANES), 1)
    picked = jnp.sum(jnp.where(lane == labels_ref[...], logits, 0.0), axis=1)
    loss_ref[0, 0] = jnp.mean(lse - picked)


def kernel(input_ids, labels, emb_table, W, b):
    batch = input_ids.shape[0]
    vocab, dim = emb_table.shape
    num_labels = W.shape[0]
    assert 2 * dim == _LANES

    idx = input_ids[:, 0]
    table2 = emb_table.reshape(vocab // 2, _LANES)
    pairs = _make_sc_gather(vocab // 2, batch)(idx >> 1, table2)

    wt = jnp.zeros((dim, _LANES), jnp.float32).at[:, :num_labels].set(W.T)
    b_pad = jnp.full((1, _LANES), _NEG, jnp.float32).at[0, :num_labels].set(b)

    logits_pad, loss = pl.pallas_call(
        _cls_body,
        out_shape=(
            jax.ShapeDtypeStruct((batch, _LANES), jnp.float32),
            jax.ShapeDtypeStruct((1, 1), jnp.float32),
        ),
        in_specs=[pl.BlockSpec(memory_space=pltpu.VMEM)] * 5,
        out_specs=(
            pl.BlockSpec(memory_space=pltpu.VMEM),
            pl.BlockSpec(memory_space=pltpu.SMEM),
        ),
    )(pairs, (idx & 1)[:, None], wt, b_pad, labels[:, None].astype(jnp.int32))

    return loss[0, 0], logits_pad[:, :num_labels]
